# baseline (speedup 1.0000x reference)
import functools

import jax
import jax.numpy as jnp
from jax import lax
from jax.experimental import pallas as pl
from jax.experimental.pallas import tpu as pltpu
from jax.experimental.pallas import tpu_sc as plsc

NUM_CHARS = 96
MAXW = 11
MAXL = MAXW + 1
BSZ, SEQ, NUM_WORDS = 4, 8, 1024
NBS = BSZ * SEQ
NWORKERS = 32
WPW = NUM_WORDS // NWORKERS
LANES = 16
GROUPS = WPW // LANES
NTASK = NBS * GROUPS
D = MAXW + 2
RS = D * LANES


def _dl_body(x_hbm, wt_hbm, wl_hbm, out_hbm, x_v, wt_v, wl_v, swl_v, dtab,
             res_v):
    wid = lax.axis_index("s") * 2 + lax.axis_index("c")
    pltpu.sync_copy(x_hbm, x_v)
    pltpu.sync_copy(wt_hbm.at[wid], wt_v)
    pltpu.sync_copy(wl_hbm.at[wid], wl_v)
    lanes = lax.broadcasted_iota(jnp.int32, (LANES,), 0)
    zi = jnp.zeros((LANES,), jnp.int32)
    lanesj = [lanes + LANES * j for j in range(MAXL)]

    for bs in range(NBS):
        xvecf = x_v[bs, :].astype(jnp.float32)
        m = jnp.max(xvecf)
        swl_v[bs, :] = plsc.all_reduce_ffs(xvecf == jnp.full((LANES,), m))

    for g in range(GROUPS):
        goff = g * LANES
        wl_vec = wl_v[pl.ds(goff, LANES)]
        wch = [wt_v[jj, pl.ds(goff, LANES)] for jj in range(MAXW)]
        wlf = wl_vec.astype(jnp.float32)
        row1 = [jnp.where(c - 1 <= wl_vec, jnp.float32(-2),
                          jnp.float32(-(c + 1))) for c in range(1, D)]
        outl = (wl_vec + 1) * LANES + lanes

        def task_body(bs, carry):
            swl_vec = swl_v[bs, :]
            swl_s = swl_vec[0]
            maxdist = wlf + swl_vec.astype(jnp.float32)

            for r in range(D):
                dtab[pl.ds(r * RS, LANES)] = maxdist - jnp.float32(r)
            for c in range(1, D):
                dtab[pl.ds(c * LANES, LANES)] = maxdist - jnp.float32(c)
            for c in range(1, D):
                dtab[pl.ds(RS + c * LANES, LANES)] = row1[c - 1]
            for r in range(2, D):
                dtab[pl.ds(r * RS + LANES, LANES)] = jnp.where(
                    r - 1 <= swl_vec, jnp.float32(-2), jnp.float32(-(r + 1)))

            bsv = jnp.full((LANES,), bs)

            def row_body(i, kd16):
                kd16 = list(kd16)
                xcv = plsc.load_gather(x_v, [bsv, jnp.full((LANES,), i - 1)])
                fi = jnp.full((LANES,), i).astype(jnp.float32)
                prev = jnp.where(i <= swl_vec, jnp.float32(-2), -(fi + 2.0))
                row = i * RS
                idv = jnp.full((LANES,), row)
                topleft = dtab[pl.ds(row + LANES, LANES)]
                dbil = lanes
                for j in range(1, MAXL):
                    top = dtab[pl.ds(row + (j + 1) * LANES, LANES)]
                    meq = wch[j - 1] == xcv
                    dt = plsc.load_gather(dtab, [kd16[j] + dbil])
                    c3 = topleft + jnp.where(meq, jnp.float32(-2),
                                             jnp.float32(-1))
                    val = jnp.minimum(jnp.minimum(jnp.minimum(top, c3),
                                                  dt - 3.0), prev)
                    dtab[pl.ds(row + RS + (j + 1) * LANES, LANES)] = val
                    kd16[j] = jnp.where(meq, idv, kd16[j])
                    dbil = jnp.where(meq, lanesj[j], dbil)
                    prev = val
                    topleft = top
                return tuple(kd16)

            lax.fori_loop(1, swl_s + 1, row_body, tuple([zi] * MAXL))

            outv = plsc.load_gather(dtab, [(swl_vec + 1) * RS + outl])
            res_v[bs * GROUPS + g, :] = outv + maxdist + 2.0
            return carry

        lax.fori_loop(0, NBS, task_body, 0)

    pltpu.sync_copy(res_v, out_hbm.at[wid])


@functools.lru_cache(maxsize=1)
def _build():
    mesh = plsc.VectorSubcoreMesh(
        core_axis_name="c", subcore_axis_name="s", num_cores=2, num_subcores=16)
    return pl.kernel(
        _dl_body,
        out_type=jax.ShapeDtypeStruct((NWORKERS, NTASK, LANES), jnp.float32),
        mesh=mesh,
        scratch_types=[
            pltpu.VMEM((NBS, LANES), jnp.int32),
            pltpu.VMEM((MAXW, WPW), jnp.int32),
            pltpu.VMEM((WPW,), jnp.int32),
            pltpu.VMEM((NBS, LANES), jnp.int32),
            pltpu.VMEM((D * D * LANES,), jnp.float32),
            pltpu.VMEM((NTASK, LANES), jnp.float32),
        ],
        compiler_params=pltpu.CompilerParams(needs_layout_passes=False),
    )


def kernel(x, words, word_lengths):
    xf = x.reshape(NBS, MAXL)
    xf = jnp.pad(xf, ((0, 0), (0, LANES - MAXL)), constant_values=-1)
    wt = words.T.reshape(MAXW, NWORKERS, WPW).transpose(1, 0, 2)
    wlc = word_lengths.reshape(NWORKERS, WPW)
    out = _build()(xf, wt, wlc)
    out = out.reshape(NWORKERS, NBS, GROUPS, LANES)
    out = out.transpose(1, 0, 2, 3).reshape(BSZ, SEQ, NUM_WORDS)
    return out


# speedup vs baseline: 1.0203x; 1.0203x over previous
import functools

import jax
import jax.numpy as jnp
from jax import lax
from jax.experimental import pallas as pl
from jax.experimental.pallas import tpu as pltpu
from jax.experimental.pallas import tpu_sc as plsc

NUM_CHARS = 96
MAXW = 11
MAXL = MAXW + 1
BSZ, SEQ, NUM_WORDS = 4, 8, 1024
NBS = BSZ * SEQ
NWORKERS = 32
WPW = NUM_WORDS // NWORKERS
LANES = 16
GROUPS = WPW // LANES
NTASK = NBS * GROUPS
D = MAXW + 2
RS = D * LANES


def _dl_body(x_hbm, wt_hbm, wl_hbm, out_hbm, x_v, wt_v, wl_v, swl_v, dtab,
             res_v):
    wid = lax.axis_index("s") * 2 + lax.axis_index("c")
    pltpu.sync_copy(x_hbm, x_v)
    pltpu.sync_copy(wt_hbm.at[wid], wt_v)
    pltpu.sync_copy(wl_hbm.at[wid], wl_v)
    lanes = lax.broadcasted_iota(jnp.int32, (LANES,), 0)
    zi = jnp.zeros((LANES,), jnp.int32)
    lanesj = [lanes + LANES * j for j in range(MAXL)]

    for bs in range(NBS):
        xvecf = x_v[bs, :].astype(jnp.float32)
        m = jnp.max(xvecf)
        swl_v[bs, :] = plsc.all_reduce_ffs(xvecf == jnp.full((LANES,), m))

    big = jnp.full((LANES,), 1e9, jnp.float32)
    mtwo = jnp.full((LANES,), -2.0, jnp.float32)
    for r in range(D):
        dtab[pl.ds(r * RS, LANES)] = big
    for c in range(1, D):
        dtab[pl.ds(c * LANES, LANES)] = big
    for r in range(1, D):
        dtab[pl.ds(r * RS + LANES, LANES)] = mtwo

    for g in range(GROUPS):
        goff = g * LANES
        wl_vec = wl_v[pl.ds(goff, LANES)]
        wch = [wt_v[jj, pl.ds(goff, LANES)] for jj in range(MAXW)]
        wlf = wl_vec.astype(jnp.float32)
        row1 = [jnp.where(c - 1 <= wl_vec, jnp.float32(-2),
                          jnp.float32(-(c + 1))) for c in range(1, D)]
        outl = (wl_vec + 1) * LANES + lanes

        for c in range(2, D):
            dtab[pl.ds(RS + c * LANES, LANES)] = row1[c - 1]

        def task_body(bs, carry):
            swl_vec = swl_v[bs, :]
            swl_s = swl_vec[0]
            maxdist = wlf + swl_vec.astype(jnp.float32)
            bsv = jnp.full((LANES,), bs)

            def row_body(i, kd16):
                kd16 = list(kd16)
                xcv = plsc.load_gather(x_v, [bsv, jnp.full((LANES,), i - 1)])
                prev = mtwo
                row = i * RS
                idv = jnp.full((LANES,), row)
                topleft = mtwo
                dbil = lanes
                for j in range(1, MAXL):
                    top = dtab[pl.ds(row + (j + 1) * LANES, LANES)]
                    meq = wch[j - 1] == xcv
                    dt = plsc.load_gather(dtab, [kd16[j] + dbil])
                    c3 = topleft + jnp.where(meq, jnp.float32(-2),
                                             jnp.float32(-1))
                    val = jnp.minimum(jnp.minimum(jnp.minimum(top, c3),
                                                  dt - 3.0), prev)
                    dtab[pl.ds(row + RS + (j + 1) * LANES, LANES)] = val
                    kd16[j] = jnp.where(meq, idv, kd16[j])
                    dbil = jnp.where(meq, lanesj[j], dbil)
                    prev = val
                    topleft = top
                return tuple(kd16)

            lax.fori_loop(1, swl_s + 1, row_body, tuple([zi] * MAXL))

            outv = plsc.load_gather(dtab, [(swl_vec + 1) * RS + outl])
            res_v[bs * GROUPS + g, :] = outv + maxdist + 2.0
            return carry

        lax.fori_loop(0, NBS, task_body, 0)

    pltpu.sync_copy(res_v, out_hbm.at[wid])


@functools.lru_cache(maxsize=1)
def _build():
    mesh = plsc.VectorSubcoreMesh(
        core_axis_name="c", subcore_axis_name="s", num_cores=2, num_subcores=16)
    return pl.kernel(
        _dl_body,
        out_type=jax.ShapeDtypeStruct((NWORKERS, NTASK, LANES), jnp.float32),
        mesh=mesh,
        scratch_types=[
            pltpu.VMEM((NBS, LANES), jnp.int32),
            pltpu.VMEM((MAXW, WPW), jnp.int32),
            pltpu.VMEM((WPW,), jnp.int32),
            pltpu.VMEM((NBS, LANES), jnp.int32),
            pltpu.VMEM((D * D * LANES,), jnp.float32),
            pltpu.VMEM((NTASK, LANES), jnp.float32),
        ],
        compiler_params=pltpu.CompilerParams(needs_layout_passes=False),
    )


def kernel(x, words, word_lengths):
    xf = x.reshape(NBS, MAXL)
    xf = jnp.pad(xf, ((0, 0), (0, LANES - MAXL)), constant_values=-1)
    wt = words.T.reshape(MAXW, NWORKERS, WPW).transpose(1, 0, 2)
    wlc = word_lengths.reshape(NWORKERS, WPW)
    out = _build()(xf, wt, wlc)
    out = out.reshape(NWORKERS, NBS, GROUPS, LANES)
    out = out.transpose(1, 0, 2, 3).reshape(BSZ, SEQ, NUM_WORDS)
    return out


# speedup vs baseline: 1.1203x; 1.0980x over previous
import functools

import jax
import jax.numpy as jnp
from jax import lax
from jax.experimental import pallas as pl
from jax.experimental.pallas import tpu as pltpu
from jax.experimental.pallas import tpu_sc as plsc

NUM_CHARS = 96
MAXW = 11
MAXL = MAXW + 1
BSZ, SEQ, NUM_WORDS = 4, 8, 1024
NBS = BSZ * SEQ
NWORKERS = 32
WPW = NUM_WORDS // NWORKERS
LANES = 16
GROUPS = WPW // LANES
NTASK = NBS * GROUPS
D = MAXW + 2
RS = D * LANES


def _dl_body(x_hbm, wt_hbm, wl_hbm, out_hbm, x_v, wt_v, wl_v, swl_v, dtab,
             res_v):
    wid = lax.axis_index("s") * 2 + lax.axis_index("c")
    pltpu.sync_copy(x_hbm, x_v.at[pl.ds(0, NBS * MAXL)])
    pltpu.sync_copy(wt_hbm.at[pl.ds(wid * WPW, WPW)], wt_v)
    pltpu.sync_copy(wl_hbm.at[pl.ds(wid * WPW, WPW)], wl_v)
    lanes = lax.broadcasted_iota(jnp.int32, (LANES,), 0)
    zi = jnp.zeros((LANES,), jnp.int32)
    lanesj = [lanes + LANES * j for j in range(MAXL)]
    mask12 = lanes < jnp.int32(MAXL)

    for bs in range(NBS):
        xvecf = jnp.where(mask12,
                          x_v[pl.ds(bs * MAXL, LANES)].astype(jnp.float32),
                          jnp.float32(-1))
        m = jnp.max(xvecf)
        swl_v[bs, :] = plsc.all_reduce_ffs(xvecf == jnp.full((LANES,), m))

    big = jnp.full((LANES,), 1e9, jnp.float32)
    mtwo = jnp.full((LANES,), -2.0, jnp.float32)
    for r in range(D):
        dtab[pl.ds(r * RS, LANES)] = big
    for c in range(1, D):
        dtab[pl.ds(c * LANES, LANES)] = big
    for r in range(1, D):
        dtab[pl.ds(r * RS + LANES, LANES)] = mtwo

    for g in range(GROUPS):
        goff = g * LANES
        wl_vec = wl_v[pl.ds(goff, LANES)]
        goffl = lanes + goff
        wch = [plsc.load_gather(wt_v, [goffl, jnp.full((LANES,), jj)])
               for jj in range(MAXW)]
        wlf = wl_vec.astype(jnp.float32)
        row1 = [jnp.where(c - 1 <= wl_vec, jnp.float32(-2),
                          jnp.float32(-(c + 1))) for c in range(1, D)]
        outl = (wl_vec + 1) * LANES + lanes

        for c in range(2, D):
            dtab[pl.ds(RS + c * LANES, LANES)] = row1[c - 1]

        def task_body(bs, carry):
            swl_vec = swl_v[bs, :]
            swl_s = swl_vec[0]
            maxdist = wlf + swl_vec.astype(jnp.float32)
            xbase = bs * MAXL - 1

            def row_body(i, kd16):
                kd16 = list(kd16)
                xcv = plsc.load_gather(x_v, [jnp.full((LANES,), xbase + i)])
                prev = mtwo
                row = i * RS
                idv = jnp.full((LANES,), row)
                topleft = mtwo
                dbil = lanes
                for j in range(1, MAXL):
                    top = dtab[pl.ds(row + (j + 1) * LANES, LANES)]
                    meq = wch[j - 1] == xcv
                    dt = plsc.load_gather(dtab, [kd16[j] + dbil])
                    c3 = topleft + jnp.where(meq, jnp.float32(-2),
                                             jnp.float32(-1))
                    val = jnp.minimum(jnp.minimum(jnp.minimum(top, c3),
                                                  dt - 3.0), prev)
                    dtab[pl.ds(row + RS + (j + 1) * LANES, LANES)] = val
                    kd16[j] = jnp.where(meq, idv, kd16[j])
                    dbil = jnp.where(meq, lanesj[j], dbil)
                    prev = val
                    topleft = top
                return tuple(kd16)

            lax.fori_loop(1, swl_s + 1, row_body, tuple([zi] * MAXL))

            outv = plsc.load_gather(dtab, [(swl_vec + 1) * RS + outl])
            res_v[bs, pl.ds(goff, LANES)] = outv + maxdist + 2.0
            return carry

        lax.fori_loop(0, NBS, task_body, 0)

    pltpu.sync_copy(res_v, out_hbm.at[:, pl.ds(wid * WPW, WPW)])


@functools.lru_cache(maxsize=1)
def _build():
    mesh = plsc.VectorSubcoreMesh(
        core_axis_name="c", subcore_axis_name="s", num_cores=2, num_subcores=16)
    return pl.kernel(
        _dl_body,
        out_type=jax.ShapeDtypeStruct((NBS, NUM_WORDS), jnp.float32),
        mesh=mesh,
        scratch_types=[
            pltpu.VMEM((NBS * MAXL + LANES,), jnp.int32),
            pltpu.VMEM((WPW, MAXW), jnp.int32),
            pltpu.VMEM((WPW,), jnp.int32),
            pltpu.VMEM((NBS, LANES), jnp.int32),
            pltpu.VMEM((D * D * LANES,), jnp.float32),
            pltpu.VMEM((NBS, WPW), jnp.float32),
        ],
        compiler_params=pltpu.CompilerParams(
            needs_layout_passes=False, use_tc_tiling_on_sc=False),
    )


def kernel(x, words, word_lengths):
    out = _build()(x.reshape(-1), words, word_lengths)
    return out.reshape(BSZ, SEQ, NUM_WORDS)
